# baseline (device time: 216368 ns/iter reference)
import jax
import jax.numpy as jnp
from jax import lax
from jax.experimental import pallas as pl
from jax.experimental.pallas import tpu as pltpu

N_DEV = 16
DH = 128
SCALE = 0.08838834764831843


def _rope(t, cos, sin):
    b, s, h, d = t.shape
    t2 = t.reshape(b, s, h, d // 2, 2)
    t_r = jnp.stack([-t2[..., 1], t2[..., 0]], axis=-1).reshape(b, s, h, d)
    return t * cos[None, :, None, :] + t_r * sin[None, :, None, :]


def _local_partial(x, Wq, Wk, Wv, Wo):
    B, Sq, D = x.shape
    Hl = Wq.shape[1] // DH
    xb = x.astype(jnp.bfloat16).reshape(B * Sq, D)

    def proj(W):
        y = jnp.dot(xb, W.astype(jnp.bfloat16),
                    preferred_element_type=jnp.float32)
        return y.reshape(B, Sq, Hl, DH)

    Q, K, V = proj(Wq), proj(Wk), proj(Wv)

    inv = 1.0 / (10000.0 ** (jnp.arange(0, DH, 2, dtype=jnp.float32) / DH))
    pos = jnp.arange(Sq, dtype=jnp.float32)[:, None] * inv[None, :]
    cos = jnp.repeat(jnp.cos(pos), 2, axis=-1)
    sin = jnp.repeat(jnp.sin(pos), 2, axis=-1)
    Q = _rope(Q, cos, sin)
    K = _rope(K, cos, sin)

    s = jnp.einsum(
        "bihd,bjhd->bhij",
        Q.astype(jnp.bfloat16), K.astype(jnp.bfloat16),
        preferred_element_type=jnp.float32,
    ) * SCALE
    w = jax.nn.softmax(s, axis=-1)
    ctx = jnp.einsum(
        "bhij,bjhd->bihd",
        w.astype(jnp.bfloat16), V.astype(jnp.bfloat16),
        preferred_element_type=jnp.float32,
    ).reshape(B * Sq, Hl * DH)
    return jnp.dot(ctx.astype(jnp.bfloat16), Wo.astype(jnp.bfloat16),
                   preferred_element_type=jnp.float32)


def _ring_allreduce(p):
    M, N = p.shape
    C = M // N_DEV

    def body(in_ref, out_ref, comm_ref, rs_send, rs_recv, ag_send, ag_recv):
        me = lax.axis_index("i")
        left = (me - 1) % N_DEV
        right = (me + 1) % N_DEV

        barrier = pltpu.get_barrier_semaphore()
        for nbr in (left, right):
            pl.semaphore_signal(barrier, inc=1, device_id=(nbr,),
                                device_id_type=pl.DeviceIdType.MESH)
        pl.semaphore_wait(barrier, 2)

        out_ref[...] = in_ref[...]

        for s in range(N_DEV - 1):
            send_idx = (me - s) % N_DEV
            recv_idx = (me - s - 1) % N_DEV
            rdma = pltpu.make_async_remote_copy(
                src_ref=out_ref.at[pl.ds(send_idx * C, C), :],
                dst_ref=comm_ref.at[s],
                send_sem=rs_send.at[s],
                recv_sem=rs_recv.at[s],
                device_id=(right,),
                device_id_type=pl.DeviceIdType.MESH,
            )
            rdma.start()
            rdma.wait()
            rows = pl.ds(recv_idx * C, C)
            out_ref[rows, :] = out_ref[rows, :] + comm_ref[s]

        for s in range(N_DEV - 1):
            idx = (me + 1 - s) % N_DEV
            rows = pl.ds(idx * C, C)
            rdma = pltpu.make_async_remote_copy(
                src_ref=out_ref.at[rows, :],
                dst_ref=out_ref.at[rows, :],
                send_sem=ag_send.at[s],
                recv_sem=ag_recv.at[s],
                device_id=(right,),
                device_id_type=pl.DeviceIdType.MESH,
            )
            rdma.start()
            rdma.wait()

        def _exit(second_barrier):
            for nbr in (left, right):
                pl.semaphore_signal(second_barrier, inc=1, device_id=(nbr,),
                                    device_id_type=pl.DeviceIdType.MESH)
            pl.semaphore_wait(second_barrier, 2)

        pl.run_scoped(_exit, second_barrier=pltpu.SemaphoreType.REGULAR)

    return pl.pallas_call(
        body,
        out_shape=jax.ShapeDtypeStruct((M, N), jnp.float32),
        in_specs=[pl.BlockSpec(memory_space=pltpu.VMEM)],
        out_specs=pl.BlockSpec(memory_space=pltpu.VMEM),
        scratch_shapes=[
            pltpu.VMEM((N_DEV - 1, C, N), jnp.float32),
            pltpu.SemaphoreType.DMA((N_DEV - 1,)),
            pltpu.SemaphoreType.DMA((N_DEV - 1,)),
            pltpu.SemaphoreType.DMA((N_DEV - 1,)),
            pltpu.SemaphoreType.DMA((N_DEV - 1,)),
        ],
        compiler_params=pltpu.CompilerParams(collective_id=0),
    )(p)


def kernel(x, Wq, Wk, Wv, Wo):
    B, Sq, D = x.shape
    part = _local_partial(x, Wq, Wk, Wv, Wo)
    return _ring_allreduce(part).reshape(B, Sq, D)


# device time: 137422 ns/iter; 1.5745x vs baseline; 1.5745x over previous
import jax
import jax.numpy as jnp
from jax import lax
from jax.experimental import pallas as pl
from jax.experimental.pallas import tpu as pltpu

N_DEV = 16
DH = 128
SCALE = 0.08838834764831843


def _rope(t, cos, sin):
    b, s, h, d = t.shape
    t2 = t.reshape(b, s, h, d // 2, 2)
    t_r = jnp.stack([-t2[..., 1], t2[..., 0]], axis=-1).reshape(b, s, h, d)
    return t * cos[None, :, None, :] + t_r * sin[None, :, None, :]


def _local_partial(x, Wq, Wk, Wv, Wo):
    B, Sq, D = x.shape
    Hl = Wq.shape[1] // DH
    xb = x.astype(jnp.bfloat16).reshape(B * Sq, D)

    def proj(W):
        y = jnp.dot(xb, W.astype(jnp.bfloat16),
                    preferred_element_type=jnp.float32)
        return y.reshape(B, Sq, Hl, DH)

    Q, K, V = proj(Wq), proj(Wk), proj(Wv)

    inv = 1.0 / (10000.0 ** (jnp.arange(0, DH, 2, dtype=jnp.float32) / DH))
    pos = jnp.arange(Sq, dtype=jnp.float32)[:, None] * inv[None, :]
    cos = jnp.repeat(jnp.cos(pos), 2, axis=-1)
    sin = jnp.repeat(jnp.sin(pos), 2, axis=-1)
    Q = _rope(Q, cos, sin)
    K = _rope(K, cos, sin)

    s = jnp.einsum(
        "bihd,bjhd->bhij",
        Q.astype(jnp.bfloat16), K.astype(jnp.bfloat16),
        preferred_element_type=jnp.float32,
    ) * SCALE
    w = jax.nn.softmax(s, axis=-1)
    ctx = jnp.einsum(
        "bhij,bjhd->bihd",
        w.astype(jnp.bfloat16), V.astype(jnp.bfloat16),
        preferred_element_type=jnp.float32,
    ).reshape(B * Sq, Hl * DH)
    return jnp.dot(ctx.astype(jnp.bfloat16), Wo.astype(jnp.bfloat16),
                   preferred_element_type=jnp.float32)


LOG2_DEV = 4


def _butterfly_allreduce(p):
    M, N = p.shape

    def body(in_ref, out_ref, c0, c1, c2, c3, rs_send, rs_recv,
             ag_send, ag_recv):
        me = lax.axis_index("i")
        comms = [c0, c1, c2, c3]

        barrier = pltpu.get_barrier_semaphore()
        for k in range(LOG2_DEV):
            pl.semaphore_signal(barrier, inc=1, device_id=(me ^ (1 << k),),
                                device_id_type=pl.DeviceIdType.MESH)
        pl.semaphore_wait(barrier, LOG2_DEV)

        out_ref[...] = in_ref[...]

        off = me * 0
        for k in range(LOG2_DEV):
            half = M >> (k + 1)
            b = (me >> k) & 1
            partner = me ^ (1 << k)
            keep_off = off + b * half
            send_off = off + (1 - b) * half
            rdma = pltpu.make_async_remote_copy(
                src_ref=out_ref.at[pl.ds(send_off, half), :],
                dst_ref=comms[k].at[pl.ds(0, half), :],
                send_sem=rs_send.at[k],
                recv_sem=rs_recv.at[k],
                device_id=(partner,),
                device_id_type=pl.DeviceIdType.MESH,
            )
            rdma.start()
            rdma.wait()
            rows = pl.ds(keep_off, half)
            out_ref[rows, :] = out_ref[rows, :] + comms[k][pl.ds(0, half), :]
            off = keep_off

        for k in reversed(range(LOG2_DEV)):
            seg = M >> (k + 1)
            b = (me >> k) & 1
            partner = me ^ (1 << k)
            rows = pl.ds(off, seg)
            rdma = pltpu.make_async_remote_copy(
                src_ref=out_ref.at[rows, :],
                dst_ref=out_ref.at[rows, :],
                send_sem=ag_send.at[k],
                recv_sem=ag_recv.at[k],
                device_id=(partner,),
                device_id_type=pl.DeviceIdType.MESH,
            )
            rdma.start()
            rdma.wait()
            off = off - b * seg

        def _exit(second_barrier):
            for k in range(LOG2_DEV):
                pl.semaphore_signal(second_barrier, inc=1,
                                    device_id=(me ^ (1 << k),),
                                    device_id_type=pl.DeviceIdType.MESH)
            pl.semaphore_wait(second_barrier, LOG2_DEV)

        pl.run_scoped(_exit, second_barrier=pltpu.SemaphoreType.REGULAR)

    return pl.pallas_call(
        body,
        out_shape=jax.ShapeDtypeStruct((M, N), jnp.bfloat16),
        in_specs=[pl.BlockSpec(memory_space=pltpu.VMEM)],
        out_specs=pl.BlockSpec(memory_space=pltpu.VMEM),
        scratch_shapes=[
            pltpu.VMEM((M >> 1, N), jnp.bfloat16),
            pltpu.VMEM((M >> 2, N), jnp.bfloat16),
            pltpu.VMEM((M >> 3, N), jnp.bfloat16),
            pltpu.VMEM((M >> 4, N), jnp.bfloat16),
            pltpu.SemaphoreType.DMA((LOG2_DEV,)),
            pltpu.SemaphoreType.DMA((LOG2_DEV,)),
            pltpu.SemaphoreType.DMA((LOG2_DEV,)),
            pltpu.SemaphoreType.DMA((LOG2_DEV,)),
        ],
        compiler_params=pltpu.CompilerParams(collective_id=0),
    )(p)


def kernel(x, Wq, Wk, Wv, Wo):
    B, Sq, D = x.shape
    part = _local_partial(x, Wq, Wk, Wv, Wo)
    out = _butterfly_allreduce(part.astype(jnp.bfloat16))
    return out.astype(jnp.float32).reshape(B, Sq, D)


# device time: 96305 ns/iter; 2.2467x vs baseline; 1.4269x over previous
import jax
import jax.numpy as jnp
from jax import lax
from jax.experimental import pallas as pl
from jax.experimental.pallas import tpu as pltpu

N_DEV = 16
DH = 128
SCALE = 0.08838834764831843


def _local_partial(x, Wq, Wk, Wv, Wo):
    B, Sq, D = x.shape
    Hl = Wq.shape[1] // DH
    half = DH // 2
    xb = x.astype(jnp.bfloat16).reshape(B * Sq, D)

    def deint(W):
        return (W.astype(jnp.bfloat16)
                .reshape(D, Hl, half, 2)
                .transpose(0, 1, 3, 2)
                .reshape(D, Hl * DH))

    def proj(Wb):
        y = jnp.dot(xb, Wb, preferred_element_type=jnp.bfloat16)
        return y.reshape(B, Sq, Hl, DH).transpose(0, 2, 1, 3)

    Q = proj(deint(Wq))
    K = proj(deint(Wk))
    V = proj(Wv.astype(jnp.bfloat16))

    inv = 1.0 / (10000.0 ** (jnp.arange(0, DH, 2, dtype=jnp.float32) / DH))
    ang = jnp.arange(Sq, dtype=jnp.float32)[:, None] * inv[None, :]
    cos = jnp.cos(ang)[None, None, :, :].astype(jnp.bfloat16)
    sin = jnp.sin(ang)[None, None, :, :].astype(jnp.bfloat16)

    def rope(t):
        t1, t2 = t[..., :half], t[..., half:]
        return jnp.concatenate([t1 * cos - t2 * sin,
                                t2 * cos + t1 * sin], axis=-1)

    Q, K = rope(Q), rope(K)

    s = jnp.einsum("bhid,bhjd->bhij", Q, K,
                   preferred_element_type=jnp.float32) * SCALE
    w = jax.nn.softmax(s, axis=-1).astype(jnp.bfloat16)
    ctx = jnp.einsum("bhij,bhjd->bhid", w, V,
                     preferred_element_type=jnp.bfloat16)
    ctx = ctx.transpose(0, 2, 1, 3).reshape(B * Sq, Hl * DH)
    return jnp.dot(ctx, Wo.astype(jnp.bfloat16),
                   preferred_element_type=jnp.bfloat16)


def _radix4_allreduce(p):
    M, N = p.shape
    Q4 = M // 4
    S16 = M // 16

    def body(in_ref, out_ref, land_p, land_z,
             rsp_send, rsp_recv, rsz_send, rsz_recv,
             agz_send, agz_recv, agp_send, agp_recv):
        me = lax.axis_index("i")
        q = me & 3
        r = me >> 2

        plane_peers = [me ^ t for t in (1, 2, 3)]
        z_peers = [me ^ (t << 2) for t in (1, 2, 3)]

        barrier = pltpu.get_barrier_semaphore()
        for peer in plane_peers + z_peers:
            pl.semaphore_signal(barrier, inc=1, device_id=(peer,),
                                device_id_type=pl.DeviceIdType.MESH)
        pl.semaphore_wait(barrier, 6)

        out_ref[...] = in_ref[...]

        rdmas = []
        for t in (1, 2, 3):
            rdma = pltpu.make_async_remote_copy(
                src_ref=out_ref.at[pl.ds(((q ^ t)) * Q4, Q4), :],
                dst_ref=land_p.at[t - 1],
                send_sem=rsp_send.at[t - 1],
                recv_sem=rsp_recv.at[t - 1],
                device_id=(plane_peers[t - 1],),
                device_id_type=pl.DeviceIdType.MESH,
            )
            rdma.start()
            rdmas.append(rdma)
        myq = pl.ds(q * Q4, Q4)
        for t in (1, 2, 3):
            rdmas[t - 1].wait()
            out_ref[myq, :] = out_ref[myq, :] + land_p[t - 1]

        rdmas = []
        for t in (1, 2, 3):
            rdma = pltpu.make_async_remote_copy(
                src_ref=out_ref.at[pl.ds(q * Q4 + (r ^ t) * S16, S16), :],
                dst_ref=land_z.at[t - 1],
                send_sem=rsz_send.at[t - 1],
                recv_sem=rsz_recv.at[t - 1],
                device_id=(z_peers[t - 1],),
                device_id_type=pl.DeviceIdType.MESH,
            )
            rdma.start()
            rdmas.append(rdma)
        mine = pl.ds(q * Q4 + r * S16, S16)
        for t in (1, 2, 3):
            rdmas[t - 1].wait()
            out_ref[mine, :] = out_ref[mine, :] + land_z[t - 1]

        rdmas = []
        for t in (1, 2, 3):
            rdma = pltpu.make_async_remote_copy(
                src_ref=out_ref.at[mine, :],
                dst_ref=out_ref.at[mine, :],
                send_sem=agz_send.at[t - 1],
                recv_sem=agz_recv.at[t - 1],
                device_id=(z_peers[t - 1],),
                device_id_type=pl.DeviceIdType.MESH,
            )
            rdma.start()
            rdmas.append(rdma)
        for rdma in rdmas:
            rdma.wait()

        rdmas = []
        for t in (1, 2, 3):
            rdma = pltpu.make_async_remote_copy(
                src_ref=out_ref.at[myq, :],
                dst_ref=out_ref.at[myq, :],
                send_sem=agp_send.at[t - 1],
                recv_sem=agp_recv.at[t - 1],
                device_id=(plane_peers[t - 1],),
                device_id_type=pl.DeviceIdType.MESH,
            )
            rdma.start()
            rdmas.append(rdma)
        for rdma in rdmas:
            rdma.wait()

        def _exit(second_barrier):
            for peer in plane_peers + z_peers:
                pl.semaphore_signal(second_barrier, inc=1,
                                    device_id=(peer,),
                                    device_id_type=pl.DeviceIdType.MESH)
            pl.semaphore_wait(second_barrier, 6)

        pl.run_scoped(_exit, second_barrier=pltpu.SemaphoreType.REGULAR)

    return pl.pallas_call(
        body,
        out_shape=jax.ShapeDtypeStruct((M, N), jnp.bfloat16),
        in_specs=[pl.BlockSpec(memory_space=pltpu.VMEM)],
        out_specs=pl.BlockSpec(memory_space=pltpu.VMEM),
        scratch_shapes=[
            pltpu.VMEM((3, Q4, N), jnp.bfloat16),
            pltpu.VMEM((3, S16, N), jnp.bfloat16),
            pltpu.SemaphoreType.DMA((3,)),
            pltpu.SemaphoreType.DMA((3,)),
            pltpu.SemaphoreType.DMA((3,)),
            pltpu.SemaphoreType.DMA((3,)),
            pltpu.SemaphoreType.DMA((3,)),
            pltpu.SemaphoreType.DMA((3,)),
            pltpu.SemaphoreType.DMA((3,)),
            pltpu.SemaphoreType.DMA((3,)),
        ],
        compiler_params=pltpu.CompilerParams(collective_id=0),
    )(p)


def kernel(x, Wq, Wk, Wv, Wo):
    B, Sq, D = x.shape
    part = _local_partial(x, Wq, Wk, Wv, Wo)
    out = _radix4_allreduce(part.astype(jnp.bfloat16))
    return out.astype(jnp.float32).reshape(B, Sq, D)


# device time: 90821 ns/iter; 2.3824x vs baseline; 1.0604x over previous
import jax
import jax.numpy as jnp
from jax import lax
from jax.experimental import pallas as pl
from jax.experimental.pallas import tpu as pltpu

N_DEV = 16
DH = 128
SCALE = 0.08838834764831843


def _local_partial(x, Wq, Wk, Wv, Wo):
    B, Sq, D = x.shape
    Hl = Wq.shape[1] // DH
    half = DH // 2
    xb = x.astype(jnp.bfloat16).reshape(B * Sq, D)

    def deint(W):
        return (W.astype(jnp.bfloat16)
                .reshape(D, Hl, half, 2)
                .transpose(0, 1, 3, 2)
                .reshape(D, Hl * DH))

    def proj(Wb):
        y = jnp.dot(xb, Wb, preferred_element_type=jnp.bfloat16)
        return y.reshape(B, Sq, Hl, DH).transpose(0, 2, 1, 3)

    Q = proj(deint(Wq))
    K = proj(deint(Wk))
    V = proj(Wv.astype(jnp.bfloat16))

    inv = 1.0 / (10000.0 ** (jnp.arange(0, DH, 2, dtype=jnp.float32) / DH))
    ang = jnp.arange(Sq, dtype=jnp.float32)[:, None] * inv[None, :]
    cos = jnp.cos(ang)[None, None, :, :].astype(jnp.bfloat16)
    sin = jnp.sin(ang)[None, None, :, :].astype(jnp.bfloat16)

    def rope(t):
        t1, t2 = t[..., :half], t[..., half:]
        return jnp.concatenate([t1 * cos - t2 * sin,
                                t2 * cos + t1 * sin], axis=-1)

    Q, K = rope(Q), rope(K)

    s = jnp.einsum("bhid,bhjd->bhij", Q, K,
                   preferred_element_type=jnp.float32) * SCALE
    w = jax.nn.softmax(s, axis=-1).astype(jnp.bfloat16)
    ctx = jnp.einsum("bhij,bhjd->bhid", w, V,
                     preferred_element_type=jnp.bfloat16)
    ctx = ctx.transpose(0, 2, 1, 3).reshape(B * Sq, Hl * DH)
    return jnp.dot(ctx, Wo.astype(jnp.bfloat16),
                   preferred_element_type=jnp.bfloat16)


def _radix4_allreduce(p):
    M, N = p.shape
    Q4 = M // 4
    S16 = M // 16

    def body(in_ref, out_ref, land_p, land_z,
             rsp_send, rsp_recv, rsz_send, rsz_recv,
             agz_send, agz_recv, agp_send, agp_recv):
        MESH = pl.DeviceIdType.MESH
        me = lax.axis_index("i")
        q = me & 3
        r = me >> 2

        plane_peers = [me ^ t for t in (1, 2, 3)]
        z_peers = [me ^ (t << 2) for t in (1, 2, 3)]

        barrier = pltpu.get_barrier_semaphore()
        for peer in plane_peers + z_peers:
            pl.semaphore_signal(barrier, inc=1, device_id=(peer,),
                                device_id_type=pl.DeviceIdType.MESH)
        pl.semaphore_wait(barrier, 6)

        out_ref[...] = in_ref[...]

        rdmas = []
        for t in (1, 2, 3):
            rdma = pltpu.make_async_remote_copy(
                src_ref=out_ref.at[pl.ds(((q ^ t)) * Q4, Q4), :],
                dst_ref=land_p.at[t - 1],
                send_sem=rsp_send.at[t - 1],
                recv_sem=rsp_recv.at[t - 1],
                device_id=(plane_peers[t - 1],),
                device_id_type=pl.DeviceIdType.MESH,
            )
            rdma.start()
            rdmas.append(rdma)
        myq = pl.ds(q * Q4, Q4)
        for rdma in rdmas:
            rdma.wait()
        out_ref[myq, :] = (out_ref[myq, :] + land_p[0]
                           + land_p[1] + land_p[2])

        rdmas = []
        for t in (1, 2, 3):
            rdma = pltpu.make_async_remote_copy(
                src_ref=out_ref.at[pl.ds(q * Q4 + (r ^ t) * S16, S16), :],
                dst_ref=land_z.at[t - 1],
                send_sem=rsz_send.at[t - 1],
                recv_sem=rsz_recv.at[t - 1],
                device_id=(z_peers[t - 1],),
                device_id_type=pl.DeviceIdType.MESH,
            )
            rdma.start()
            rdmas.append(rdma)
        mine = pl.ds(q * Q4 + r * S16, S16)
        for rdma in rdmas:
            rdma.wait()
        out_ref[mine, :] = (out_ref[mine, :] + land_z[0]
                            + land_z[1] + land_z[2])

        agz = []
        for t in (1, 2, 3):
            rdma = pltpu.make_async_remote_copy(
                src_ref=out_ref.at[mine, :],
                dst_ref=out_ref.at[mine, :],
                send_sem=agz_send.at[t - 1],
                recv_sem=agz_recv.at[t - 1],
                device_id=(z_peers[t - 1],),
                device_id_type=MESH,
            )
            rdma.start()
            agz.append(rdma)

        def forward_block(u, zsub):
            rows = pl.ds(q * Q4 + zsub * S16, S16)
            started = []
            for t in (1, 2, 3):
                rdma = pltpu.make_async_remote_copy(
                    src_ref=out_ref.at[rows, :],
                    dst_ref=out_ref.at[rows, :],
                    send_sem=agp_send.at[t - 1, u],
                    recv_sem=agp_recv.at[t - 1, u],
                    device_id=(plane_peers[t - 1],),
                    device_id_type=MESH,
                )
                rdma.start()
                started.append(rdma)
            return started

        agp = forward_block(0, r)
        for u in (1, 2, 3):
            agz[u - 1].wait_recv()
            agp += forward_block(u, r ^ u)
        for rdma in agz:
            rdma.wait_send()
        for rdma in agp:
            rdma.wait()

        def _exit(second_barrier):
            for peer in plane_peers + z_peers:
                pl.semaphore_signal(second_barrier, inc=1,
                                    device_id=(peer,),
                                    device_id_type=pl.DeviceIdType.MESH)
            pl.semaphore_wait(second_barrier, 6)

        pl.run_scoped(_exit, second_barrier=pltpu.SemaphoreType.REGULAR)

    return pl.pallas_call(
        body,
        out_shape=jax.ShapeDtypeStruct((M, N), jnp.bfloat16),
        in_specs=[pl.BlockSpec(memory_space=pltpu.VMEM)],
        out_specs=pl.BlockSpec(memory_space=pltpu.VMEM),
        scratch_shapes=[
            pltpu.VMEM((3, Q4, N), jnp.bfloat16),
            pltpu.VMEM((3, S16, N), jnp.bfloat16),
            pltpu.SemaphoreType.DMA((3,)),
            pltpu.SemaphoreType.DMA((3,)),
            pltpu.SemaphoreType.DMA((3,)),
            pltpu.SemaphoreType.DMA((3,)),
            pltpu.SemaphoreType.DMA((3,)),
            pltpu.SemaphoreType.DMA((3,)),
            pltpu.SemaphoreType.DMA((3, 4)),
            pltpu.SemaphoreType.DMA((3, 4)),
        ],
        compiler_params=pltpu.CompilerParams(collective_id=0),
    )(p)


def kernel(x, Wq, Wk, Wv, Wo):
    B, Sq, D = x.shape
    part = _local_partial(x, Wq, Wk, Wv, Wo)
    out = _radix4_allreduce(part.astype(jnp.bfloat16))
    return out.astype(jnp.float32).reshape(B, Sq, D)


# device time: 90714 ns/iter; 2.3852x vs baseline; 1.0012x over previous
import jax
import jax.numpy as jnp
from jax import lax
from jax.experimental import pallas as pl
from jax.experimental.pallas import tpu as pltpu

N_DEV = 16
DH = 128
SCALE = 0.08838834764831843


def _local_partial(x, Wq, Wk, Wv, Wo):
    B, Sq, D = x.shape
    Hl = Wq.shape[1] // DH
    half = DH // 2
    xb = x.astype(jnp.bfloat16).reshape(B * Sq, D)

    def deint(W):
        return (W.astype(jnp.bfloat16)
                .reshape(D, Hl, half, 2)
                .transpose(0, 1, 3, 2)
                .reshape(D, Hl * DH))

    def proj(Wb):
        y = jnp.dot(xb, Wb, preferred_element_type=jnp.bfloat16)
        return y.reshape(B, Sq, Hl, DH).transpose(0, 2, 1, 3)

    Q = proj(deint(Wq))
    K = proj(deint(Wk))
    V = proj(Wv.astype(jnp.bfloat16))

    inv = 1.0 / (10000.0 ** (jnp.arange(0, DH, 2, dtype=jnp.float32) / DH))
    ang = jnp.arange(Sq, dtype=jnp.float32)[:, None] * inv[None, :]
    cos = jnp.cos(ang)[None, None, :, :].astype(jnp.bfloat16)
    sin = jnp.sin(ang)[None, None, :, :].astype(jnp.bfloat16)

    def rope(t):
        t1, t2 = t[..., :half], t[..., half:]
        return jnp.concatenate([t1 * cos - t2 * sin,
                                t2 * cos + t1 * sin], axis=-1)

    Q, K = rope(Q), rope(K)

    s = jnp.einsum("bhid,bhjd->bhij", Q, K,
                   preferred_element_type=jnp.float32) * SCALE
    w = jax.nn.softmax(s, axis=-1).astype(jnp.bfloat16)
    ctx = jnp.einsum("bhij,bhjd->bhid", w, V,
                     preferred_element_type=jnp.bfloat16)
    return ctx.transpose(0, 2, 1, 3).reshape(B * Sq, Hl * DH)


def _radix4_allreduce(ctx, wo):
    M, N = ctx.shape
    Q4 = M // 4
    S16 = M // 16

    def body(ctx_ref, wo_ref, out_ref, land_p, land_z,
             rsp_send, rsp_recv, rsz_send, rsz_recv,
             agz_send, agz_recv, agp_send, agp_recv):
        MESH = pl.DeviceIdType.MESH
        me = lax.axis_index("i")
        q = me & 3
        r = me >> 2

        plane_peers = [me ^ t for t in (1, 2, 3)]
        z_peers = [me ^ (t << 2) for t in (1, 2, 3)]

        barrier = pltpu.get_barrier_semaphore()
        for peer in plane_peers + z_peers:
            pl.semaphore_signal(barrier, inc=1, device_id=(peer,),
                                device_id_type=pl.DeviceIdType.MESH)
        out_ref[...] = jnp.dot(
            ctx_ref[...], wo_ref[...],
            preferred_element_type=jnp.float32,
        ).astype(jnp.bfloat16)
        pl.semaphore_wait(barrier, 6)

        rdmas = []
        for t in (1, 2, 3):
            rdma = pltpu.make_async_remote_copy(
                src_ref=out_ref.at[pl.ds(((q ^ t)) * Q4, Q4), :],
                dst_ref=land_p.at[t - 1],
                send_sem=rsp_send.at[t - 1],
                recv_sem=rsp_recv.at[t - 1],
                device_id=(plane_peers[t - 1],),
                device_id_type=pl.DeviceIdType.MESH,
            )
            rdma.start()
            rdmas.append(rdma)
        myq = pl.ds(q * Q4, Q4)
        for rdma in rdmas:
            rdma.wait()
        out_ref[myq, :] = (out_ref[myq, :] + land_p[0]
                           + land_p[1] + land_p[2])

        rdmas = []
        for t in (1, 2, 3):
            rdma = pltpu.make_async_remote_copy(
                src_ref=out_ref.at[pl.ds(q * Q4 + (r ^ t) * S16, S16), :],
                dst_ref=land_z.at[t - 1],
                send_sem=rsz_send.at[t - 1],
                recv_sem=rsz_recv.at[t - 1],
                device_id=(z_peers[t - 1],),
                device_id_type=pl.DeviceIdType.MESH,
            )
            rdma.start()
            rdmas.append(rdma)
        mine = pl.ds(q * Q4 + r * S16, S16)
        for rdma in rdmas:
            rdma.wait()
        out_ref[mine, :] = (out_ref[mine, :] + land_z[0]
                            + land_z[1] + land_z[2])

        agz = []
        for t in (1, 2, 3):
            rdma = pltpu.make_async_remote_copy(
                src_ref=out_ref.at[mine, :],
                dst_ref=out_ref.at[mine, :],
                send_sem=agz_send.at[t - 1],
                recv_sem=agz_recv.at[t - 1],
                device_id=(z_peers[t - 1],),
                device_id_type=MESH,
            )
            rdma.start()
            agz.append(rdma)

        def forward_block(u, zsub):
            rows = pl.ds(q * Q4 + zsub * S16, S16)
            started = []
            for t in (1, 2, 3):
                rdma = pltpu.make_async_remote_copy(
                    src_ref=out_ref.at[rows, :],
                    dst_ref=out_ref.at[rows, :],
                    send_sem=agp_send.at[t - 1, u],
                    recv_sem=agp_recv.at[t - 1, u],
                    device_id=(plane_peers[t - 1],),
                    device_id_type=MESH,
                )
                rdma.start()
                started.append(rdma)
            return started

        agp = forward_block(0, r)
        for u in (1, 2, 3):
            agz[u - 1].wait_recv()
            agp += forward_block(u, r ^ u)
        for rdma in agz:
            rdma.wait_send()
        for rdma in agp:
            rdma.wait()

        def _exit(second_barrier):
            for peer in plane_peers + z_peers:
                pl.semaphore_signal(second_barrier, inc=1,
                                    device_id=(peer,),
                                    device_id_type=pl.DeviceIdType.MESH)
            pl.semaphore_wait(second_barrier, 6)

        pl.run_scoped(_exit, second_barrier=pltpu.SemaphoreType.REGULAR)

    return pl.pallas_call(
        body,
        out_shape=jax.ShapeDtypeStruct((M, N), jnp.bfloat16),
        in_specs=[pl.BlockSpec(memory_space=pltpu.VMEM),
                  pl.BlockSpec(memory_space=pltpu.VMEM)],
        out_specs=pl.BlockSpec(memory_space=pltpu.VMEM),
        scratch_shapes=[
            pltpu.VMEM((3, Q4, N), jnp.bfloat16),
            pltpu.VMEM((3, S16, N), jnp.bfloat16),
            pltpu.SemaphoreType.DMA((3,)),
            pltpu.SemaphoreType.DMA((3,)),
            pltpu.SemaphoreType.DMA((3,)),
            pltpu.SemaphoreType.DMA((3,)),
            pltpu.SemaphoreType.DMA((3,)),
            pltpu.SemaphoreType.DMA((3,)),
            pltpu.SemaphoreType.DMA((3, 4)),
            pltpu.SemaphoreType.DMA((3, 4)),
        ],
        compiler_params=pltpu.CompilerParams(collective_id=0),
    )(ctx, wo)


def kernel(x, Wq, Wk, Wv, Wo):
    B, Sq, D = x.shape
    ctx = _local_partial(x, Wq, Wk, Wv, Wo)
    out = _radix4_allreduce(ctx, Wo.astype(jnp.bfloat16))
    return out.astype(jnp.float32).reshape(B, Sq, D)
